# baseline (speedup 1.0000x reference)
import functools

import jax
import jax.numpy as jnp
from jax import lax
from jax.experimental import pallas as pl
from jax.experimental.pallas import tpu as pltpu
from jax.experimental.pallas import tpu_sc as plsc

N_NODES = 10000
N_EDGES = 320000
D = 128

NC = 2
NS = 16
NW = NC * NS
EPW = N_EDGES // NW
C = 80
NCH = EPW // C
NPAD = 10112
RPS = NPAD // NS

ROW_BLK = 1000
GRID = N_NODES // ROW_BLK


HALF_BLKS = GRID // 2


def _u_map(i):
    return (jnp.minimum(i, HALF_BLKS - 1), 0)


def _i_map(i):
    return (jnp.maximum(i - HALF_BLKS, 0), 0)


def _pick_h(u_ref, i_ref):
    return jnp.where(pl.program_id(0) < HALF_BLKS, u_ref[...], i_ref[...])


def _scale_body(u_ref, i_ref, n_ref, o_ref):
    o_ref[...] = _pick_h(u_ref, i_ref) * n_ref[...]


def _scale(user, item, norm):
    return pl.pallas_call(
        _scale_body,
        grid=(GRID,),
        in_specs=[
            pl.BlockSpec((ROW_BLK, D), _u_map),
            pl.BlockSpec((ROW_BLK, D), _i_map),
            pl.BlockSpec((ROW_BLK, 1), lambda i: (i, 0)),
        ],
        out_specs=pl.BlockSpec((ROW_BLK, D), lambda i: (i, 0)),
        out_shape=jax.ShapeDtypeStruct((N_NODES, D), jnp.float32),
    )(user, item, norm)


def _sc_body(hn_hbm, src_hbm, dst_hbm, z_hbm, out_hbm, *scratch):
    NB = 4
    sidx = scratch[0:NB]
    didx = scratch[NB:2 * NB]
    rbuf = scratch[2 * NB:3 * NB]
    acc_sh = scratch[3 * NB]
    semi = scratch[3 * NB + 1:3 * NB + 1 + NB]
    semg = scratch[3 * NB + 1 + NB:3 * NB + 1 + 2 * NB]

    cid = lax.axis_index("c")
    sid = lax.axis_index("s")
    wid = sid * NC + cid
    base = wid * EPW

    def iload(j, b):
        pltpu.async_copy(src_hbm.at[pl.ds(base + j * C, C)], sidx[b], semi[b])
        pltpu.async_copy(dst_hbm.at[pl.ds(base + j * C, C)], didx[b], semi[b])

    def iwait(j, b):
        pltpu.make_async_copy(src_hbm.at[pl.ds(base + j * C, C)], sidx[b], semi[b]).wait()
        pltpu.make_async_copy(dst_hbm.at[pl.ds(base + j * C, C)], didx[b], semi[b]).wait()

    def gather(b):
        pltpu.async_copy(hn_hbm.at[sidx[b]], rbuf[b], semg[b])

    def gwait(b):
        pltpu.make_async_copy(hn_hbm.at[sidx[b]], rbuf[b], semg[b]).wait()

    def scatter(b):
        pltpu.sync_copy(rbuf[b], acc_sh.at[didx[b]], add=True)

    for b in range(NB):
        iload(b, b)
    for b in range(NB):
        iwait(b, b)
        gather(b)

    pltpu.sync_copy(z_hbm, acc_sh.at[pl.ds(sid * RPS, RPS)])
    plsc.subcore_barrier()

    def body(g, _):
        j0 = g * NB
        for b in range(NB):
            gwait(b)
            scatter(b)

            @pl.when(j0 + NB + b < NCH)
            def _(b=b):
                iload(j0 + NB + b, b)

        for b in range(NB):
            @pl.when(j0 + NB + b < NCH)
            def _(b=b):
                iwait(j0 + NB + b, b)
                gather(b)

        return 0

    lax.fori_loop(0, NCH // NB, body, 0)

    gwait(0)
    scatter(0)

    plsc.subcore_barrier()
    pltpu.sync_copy(acc_sh.at[pl.ds(sid * RPS, RPS)],
                    out_hbm.at[cid, pl.ds(sid * RPS, RPS)])


_sc_segsum = functools.partial(
    pl.kernel,
    out_type=jax.ShapeDtypeStruct((NC, NPAD, D), jnp.float32),
    mesh=plsc.VectorSubcoreMesh(core_axis_name="c", subcore_axis_name="s",
                                num_cores=NC, num_subcores=NS),
    scratch_types=(
        [pltpu.VMEM((C,), jnp.int32)] * 8
        + [pltpu.VMEM((C, D), jnp.float32)] * 4
        + [pltpu.VMEM_SHARED((NPAD, D), jnp.float32)]
        + [pltpu.SemaphoreType.DMA] * 8
    ),
)(_sc_body)


def _epi_body(ap_ref, u_ref, i_ref, n_ref, w1_ref, w2_ref, o_ref):
    h = _pick_h(u_ref, i_ref)
    an = (ap_ref[0] + ap_ref[1]) * n_ref[...]
    m = (jnp.dot(an + h, w1_ref[...], preferred_element_type=jnp.float32)
         + jnp.dot(an * h, w2_ref[...], preferred_element_type=jnp.float32))
    m = jnp.where(m >= 0, m, 0.2 * m)
    nrm = jnp.sqrt(jnp.sum(m * m, axis=1, keepdims=True))
    o_ref[...] = m / jnp.maximum(nrm, 1e-12)


def _epilogue(parts, user, item, norm, W1, W2):
    return pl.pallas_call(
        _epi_body,
        grid=(GRID,),
        in_specs=[
            pl.BlockSpec((NC, ROW_BLK, D), lambda i: (0, i, 0)),
            pl.BlockSpec((ROW_BLK, D), _u_map),
            pl.BlockSpec((ROW_BLK, D), _i_map),
            pl.BlockSpec((ROW_BLK, 1), lambda i: (i, 0)),
            pl.BlockSpec((D, D), lambda i: (0, 0)),
            pl.BlockSpec((D, D), lambda i: (0, 0)),
        ],
        out_specs=pl.BlockSpec((ROW_BLK, D), lambda i: (i, 0)),
        out_shape=jax.ShapeDtypeStruct((N_NODES, D), jnp.float32),
    )(parts, user, item, norm, W1, W2)


def kernel(user_embedding, item_embedding, edge_index, norm, W1, W2):
    src = edge_index[0]
    dst = edge_index[1]
    hn = _scale(user_embedding, item_embedding, norm)
    zeros = jnp.zeros((RPS, D), jnp.float32)
    parts = _sc_segsum(hn, src, dst, zeros)
    return _epilogue(parts, user_embedding, item_embedding, norm, W1, W2)


# speedup vs baseline: 1.0304x; 1.0304x over previous
import functools

import jax
import jax.numpy as jnp
from jax import lax
from jax.experimental import pallas as pl
from jax.experimental.pallas import tpu as pltpu
from jax.experimental.pallas import tpu_sc as plsc

N_NODES = 10000
N_EDGES = 320000
D = 128

NC = 2
NS = 16
NW = NC * NS
EPW = N_EDGES // NW
C = 80
NCH = EPW // C
NPAD = 10112
RPS = NPAD // NS

ROW_BLK = 1000
GRID = N_NODES // ROW_BLK


HROWS = N_NODES // 2
EHALF = N_EDGES // 2


def _prep_body(u_ref, i_ref, n_ref, hn_ref):
    hn_ref[pl.ds(0, HROWS)] = u_ref[...] * n_ref[pl.ds(0, HROWS)]
    hn_ref[pl.ds(HROWS, HROWS)] = i_ref[...] * n_ref[pl.ds(HROWS, HROWS)]


def _prep(user, item, norm):
    return pl.pallas_call(
        _prep_body,
        out_shape=jax.ShapeDtypeStruct((N_NODES, D), jnp.float32),
    )(user, item, norm)


def _sc_body(hn_hbm, src_hbm, dst_hbm, z_hbm, out_hbm, *scratch):
    NB = 4
    sidx = scratch[0:NB]
    didx = scratch[NB:2 * NB]
    rbuf = scratch[2 * NB:3 * NB]
    acc_sh = scratch[3 * NB]
    semi = scratch[3 * NB + 1:3 * NB + 1 + NB]
    semg = scratch[3 * NB + 1 + NB:3 * NB + 1 + 2 * NB]

    cid = lax.axis_index("c")
    sid = lax.axis_index("s")
    wid = sid * NC + cid
    base = wid * EPW

    def iload(j, b):
        pltpu.async_copy(src_hbm.at[pl.ds(base + j * C, C)], sidx[b], semi[b])
        pltpu.async_copy(dst_hbm.at[pl.ds(base + j * C, C)], didx[b], semi[b])

    def iwait(j, b):
        pltpu.make_async_copy(src_hbm.at[pl.ds(base + j * C, C)], sidx[b], semi[b]).wait()
        pltpu.make_async_copy(dst_hbm.at[pl.ds(base + j * C, C)], didx[b], semi[b]).wait()

    def gather(b):
        pltpu.async_copy(hn_hbm.at[sidx[b]], rbuf[b], semg[b])

    def gwait(b):
        pltpu.make_async_copy(hn_hbm.at[sidx[b]], rbuf[b], semg[b]).wait()

    def scatter(b):
        pltpu.sync_copy(rbuf[b], acc_sh.at[didx[b]], add=True)

    for b in range(NB):
        iload(b, b)
    for b in range(NB):
        iwait(b, b)
        gather(b)

    pltpu.sync_copy(z_hbm, acc_sh.at[pl.ds(sid * RPS, RPS)])
    plsc.subcore_barrier()

    def body(g, _):
        j0 = g * NB
        for b in range(NB):
            gwait(b)
            scatter(b)

            @pl.when(j0 + NB + b < NCH)
            def _(b=b):
                iload(j0 + NB + b, b)

        for b in range(NB):
            @pl.when(j0 + NB + b < NCH)
            def _(b=b):
                iwait(j0 + NB + b, b)
                gather(b)

        return 0

    lax.fori_loop(0, NCH // NB, body, 0)

    gwait(0)
    scatter(0)

    plsc.subcore_barrier()
    pltpu.sync_copy(acc_sh.at[pl.ds(sid * RPS, RPS)],
                    out_hbm.at[cid, pl.ds(sid * RPS, RPS)])


@functools.cache
def _get_sc_segsum():
    return pl.kernel(
        _sc_body,
        out_type=jax.ShapeDtypeStruct((NC, NPAD, D), jnp.float32),
        mesh=plsc.VectorSubcoreMesh(core_axis_name="c", subcore_axis_name="s",
                                    num_cores=NC, num_subcores=NS),
        scratch_types=(
            [pltpu.VMEM((C,), jnp.int32)] * 8
            + [pltpu.VMEM((C, D), jnp.float32)] * 4
            + [pltpu.VMEM_SHARED((NPAD, D), jnp.float32)]
            + [pltpu.SemaphoreType.DMA] * 8
        ),
    )


def _epi_body(ap_ref, u_ref, i_ref, n_ref, w1_ref, w2_ref, o_ref):
    h = jnp.where(pl.program_id(0) < 1, u_ref[...], i_ref[...])
    an = (ap_ref[0] + ap_ref[1]) * n_ref[...]
    m = (jnp.dot(an + h, w1_ref[...], preferred_element_type=jnp.float32)
         + jnp.dot(an * h, w2_ref[...], preferred_element_type=jnp.float32))
    m = jnp.where(m >= 0, m, 0.2 * m)
    nrm = jnp.sqrt(jnp.sum(m * m, axis=1, keepdims=True))
    o_ref[...] = m / jnp.maximum(nrm, 1e-12)


def _epilogue(parts, user, item, norm, W1, W2):
    return pl.pallas_call(
        _epi_body,
        grid=(2,),
        in_specs=[
            pl.BlockSpec((NC, HROWS, D), lambda i: (0, i, 0)),
            pl.BlockSpec((HROWS, D), lambda i: (0, 0)),
            pl.BlockSpec((HROWS, D), lambda i: (0, 0)),
            pl.BlockSpec((HROWS, 1), lambda i: (i, 0)),
            pl.BlockSpec((D, D), lambda i: (0, 0)),
            pl.BlockSpec((D, D), lambda i: (0, 0)),
        ],
        out_specs=pl.BlockSpec((HROWS, D), lambda i: (i, 0)),
        out_shape=jax.ShapeDtypeStruct((N_NODES, D), jnp.float32),
    )(parts, user, item, norm, W1, W2)


def kernel(user_embedding, item_embedding, edge_index, norm, W1, W2):
    hn = _prep(user_embedding, item_embedding, norm)
    src = edge_index[0]
    dst = edge_index[1]
    zeros = jnp.zeros((RPS, D), jnp.float32)
    parts = _get_sc_segsum()(hn, src, dst, zeros)
    return _epilogue(parts, user_embedding, item_embedding, norm, W1, W2)


# speedup vs baseline: 1.1422x; 1.1085x over previous
import functools

import jax
import jax.numpy as jnp
from jax import lax
from jax.experimental import pallas as pl
from jax.experimental.pallas import tpu as pltpu
from jax.experimental.pallas import tpu_sc as plsc

N_NODES = 10000
N_EDGES = 320000
D = 128

NC = 2
NS = 16
NW = NC * NS
EPW = N_EDGES // NW
C = 80
NCH = EPW // C
NPAD = 10112
RPS = NPAD // NS

ROW_BLK = 1000
GRID = N_NODES // ROW_BLK


HROWS = N_NODES // 2
EHALF = N_EDGES // 2


def _prep_body(u_ref, i_ref, n_ref, hn_ref):
    hn_ref[pl.ds(0, HROWS)] = u_ref[...] * n_ref[pl.ds(0, HROWS)]
    hn_ref[pl.ds(HROWS, HROWS)] = i_ref[...] * n_ref[pl.ds(HROWS, HROWS)]


def _prep(user, item, norm):
    return pl.pallas_call(
        _prep_body,
        out_shape=jax.ShapeDtypeStruct((N_NODES, D), jnp.float32),
    )(user, item, norm)


CK = 128
NCHT = N_EDGES // CK
GMAX = (NCHT + NW - 1) // NW
NB = 2


def _sc_body(hn_hbm, e_hbm, z_hbm, out_hbm, *scratch):
    ebuf = scratch[0:NB]
    rbuf = scratch[NB:2 * NB]
    acc_sh = scratch[2 * NB]
    semi = scratch[2 * NB + 1:2 * NB + 1 + NB]
    semg = scratch[2 * NB + 1 + NB:2 * NB + 1 + 2 * NB]

    cid = lax.axis_index("c")
    sid = lax.axis_index("s")
    wid = sid * NC + cid

    def chunk(g):
        return wid + g * NW

    def iload(g, b):
        pltpu.async_copy(e_hbm.at[:, pl.ds(chunk(g) * CK, CK)], ebuf[b], semi[b])

    def iwait(g, b):
        pltpu.make_async_copy(e_hbm.at[:, pl.ds(chunk(g) * CK, CK)], ebuf[b],
                              semi[b]).wait()

    def gather(b):
        pltpu.async_copy(hn_hbm.at[ebuf[b].at[0]], rbuf[b], semg[b])

    def gwait(b):
        pltpu.make_async_copy(hn_hbm.at[ebuf[b].at[0]], rbuf[b], semg[b]).wait()

    def scatter(b):
        pltpu.sync_copy(rbuf[b], acc_sh.at[ebuf[b].at[1]], add=True)

    for b in range(NB):
        iload(b, b)
    for b in range(NB):
        iwait(b, b)
        gather(b)

    pltpu.sync_copy(z_hbm, acc_sh.at[pl.ds(sid * RPS, RPS)])
    plsc.subcore_barrier()

    def body(gg, _):
        g0 = gg * NB
        for b in range(NB):
            @pl.when(chunk(g0 + b) < NCHT)
            def _(b=b):
                gwait(b)
                scatter(b)

            @pl.when(chunk(g0 + NB + b) < NCHT)
            def _(b=b):
                iload(g0 + NB + b, b)

        for b in range(NB):
            @pl.when(chunk(g0 + NB + b) < NCHT)
            def _(b=b):
                iwait(g0 + NB + b, b)
                gather(b)

        return 0

    lax.fori_loop(0, (GMAX + NB - 1) // NB, body, 0)

    plsc.subcore_barrier()
    pltpu.sync_copy(acc_sh.at[pl.ds(sid * RPS, RPS)],
                    out_hbm.at[cid, pl.ds(sid * RPS, RPS)])


@functools.cache
def _get_sc_segsum():
    return pl.kernel(
        _sc_body,
        out_type=jax.ShapeDtypeStruct((NC, NPAD, D), jnp.float32),
        mesh=plsc.VectorSubcoreMesh(core_axis_name="c", subcore_axis_name="s",
                                    num_cores=NC, num_subcores=NS),
        scratch_types=(
            [pltpu.VMEM((2, CK), jnp.int32)] * NB
            + [pltpu.VMEM((CK, D), jnp.float32)] * NB
            + [pltpu.VMEM_SHARED((NPAD, D), jnp.float32)]
            + [pltpu.SemaphoreType.DMA] * (2 * NB)
        ),
    )


def _epi_body(ap_ref, u_ref, i_ref, n_ref, w1_ref, w2_ref, o_ref):
    h = jnp.where(pl.program_id(0) < 1, u_ref[...], i_ref[...])
    an = (ap_ref[0] + ap_ref[1]) * n_ref[...]
    m = (jnp.dot(an + h, w1_ref[...], preferred_element_type=jnp.float32)
         + jnp.dot(an * h, w2_ref[...], preferred_element_type=jnp.float32))
    m = jnp.where(m >= 0, m, 0.2 * m)
    nrm = jnp.sqrt(jnp.sum(m * m, axis=1, keepdims=True))
    o_ref[...] = m / jnp.maximum(nrm, 1e-12)


def _epilogue(parts, user, item, norm, W1, W2):
    return pl.pallas_call(
        _epi_body,
        grid=(2,),
        in_specs=[
            pl.BlockSpec((NC, HROWS, D), lambda i: (0, i, 0)),
            pl.BlockSpec((HROWS, D), lambda i: (0, 0)),
            pl.BlockSpec((HROWS, D), lambda i: (0, 0)),
            pl.BlockSpec((HROWS, 1), lambda i: (i, 0)),
            pl.BlockSpec((D, D), lambda i: (0, 0)),
            pl.BlockSpec((D, D), lambda i: (0, 0)),
        ],
        out_specs=pl.BlockSpec((HROWS, D), lambda i: (i, 0)),
        out_shape=jax.ShapeDtypeStruct((N_NODES, D), jnp.float32),
    )(parts, user, item, norm, W1, W2)


def kernel(user_embedding, item_embedding, edge_index, norm, W1, W2):
    hn = _prep(user_embedding, item_embedding, norm)
    zeros = jnp.zeros((RPS, D), jnp.float32)
    parts = _get_sc_segsum()(hn, edge_index, zeros)
    return _epilogue(parts, user_embedding, item_embedding, norm, W1, W2)
